# baseline (speedup 1.0000x reference)
import functools

import jax
import jax.numpy as jnp
from jax import lax
from jax.experimental import pallas as pl
from jax.experimental.pallas import tpu as pltpu
from jax.experimental.pallas import tpu_sc as plsc

HIDDEN = 1024
FFN = 1024
E = 64
TOPK = 2
T = 4096

S = T * TOPK
BS = 64
S_MAX = S + E * BS
NB = S_MAX // BS
BT = 512

NW = 32
APW = S // NW
NCH = 8
CH = APW // NCH
TPW = T // NW
TCH = CH // TOPK

_INTERPRET = False


def _router_body(x_ref, gw_ref, tri_ref, logits_ref, ids_ref, wts_ref,
                 wb_ref, rank_ref, counts_ref, carry_ref):
    i = pl.program_id(0)

    @pl.when(i == 0)
    def _():
        carry_ref[...] = jnp.zeros_like(carry_ref)

    x = x_ref[...]
    logits = lax.dot_general(x, gw_ref[...], (((1,), (1,)), ((), ())),
                             preferred_element_type=jnp.float32)
    logits_ref[...] = logits
    m = jnp.max(logits, axis=-1, keepdims=True)
    p = jnp.exp(logits - m)
    p = p / jnp.sum(p, axis=-1, keepdims=True)
    iot = lax.broadcasted_iota(jnp.int32, p.shape, 1)
    m0 = jnp.max(p, axis=-1, keepdims=True)
    i0 = jnp.min(jnp.where(p == m0, iot, E), axis=-1, keepdims=True)
    p2 = jnp.where(iot == i0, -jnp.inf, p)
    m1 = jnp.max(p2, axis=-1, keepdims=True)
    i1 = jnp.min(jnp.where(p2 == m1, iot, E), axis=-1, keepdims=True)
    s = m0 + m1
    ids_ref[...] = jnp.concatenate([i0, i1], axis=-1)
    wts_ref[...] = jnp.concatenate([m0 / s, m1 / s], axis=-1)
    wb_ref[...] = jnp.concatenate(
        [jnp.broadcast_to(m0 / s, (BT, 16)),
         jnp.broadcast_to(m1 / s, (BT, 16))], axis=-1)

    oh0 = (i0 == iot).astype(jnp.int32)
    oh1 = (i1 == iot).astype(jnp.int32)
    c01 = lax.dot_general(tri_ref[...], (oh0 + oh1).astype(jnp.bfloat16),
                          (((1,), (0,)), ((), ())),
                          preferred_element_type=jnp.float32
                          ).astype(jnp.int32)
    excl = c01 - oh0 - oh1
    carry = carry_ref[0:1, :]
    r0 = jnp.sum(oh0 * (excl + carry), axis=1, keepdims=True)
    r1 = jnp.sum(oh1 * (excl + oh0 + carry), axis=1, keepdims=True)
    rank_ref[...] = jnp.concatenate([r0, r1], axis=-1)
    carry_new = carry + c01[-1:, :]
    carry_ref[0:1, :] = carry_new
    counts_ref[...] = jnp.broadcast_to(carry_new, counts_ref.shape)


def _router(hidden_states, gate_w, tri):
    return pl.pallas_call(
        _router_body,
        grid=(T // BT,),
        in_specs=[
            pl.BlockSpec((BT, HIDDEN), lambda i: (i, 0)),
            pl.BlockSpec((E, HIDDEN), lambda i: (0, 0)),
            pl.BlockSpec((BT, BT), lambda i: (0, 0)),
        ],
        out_specs=[
            pl.BlockSpec((BT, E), lambda i: (i, 0)),
            pl.BlockSpec((BT, TOPK), lambda i: (i, 0)),
            pl.BlockSpec((BT, TOPK), lambda i: (i, 0)),
            pl.BlockSpec((BT, 2 * 16), lambda i: (i, 0)),
            pl.BlockSpec((BT, TOPK), lambda i: (i, 0)),
            pl.BlockSpec((8, E), lambda i: (0, 0)),
        ],
        out_shape=[
            jax.ShapeDtypeStruct((T, E), jnp.float32),
            jax.ShapeDtypeStruct((T, TOPK), jnp.int32),
            jax.ShapeDtypeStruct((T, TOPK), jnp.float32),
            jax.ShapeDtypeStruct((T, 2 * 16), jnp.float32),
            jax.ShapeDtypeStruct((T, TOPK), jnp.int32),
            jax.ShapeDtypeStruct((8, E), jnp.int32),
        ],
        scratch_shapes=[pltpu.VMEM((8, E), jnp.int32)],
        interpret=_INTERPRET,
    )(hidden_states, gate_w, tri)


def _sc_dispatch(dest3, tok3, hidden_states):
    mesh = plsc.VectorSubcoreMesh(core_axis_name="c", subcore_axis_name="s")

    @functools.partial(
        pl.kernel, mesh=mesh,
        out_type=jax.ShapeDtypeStruct((S_MAX, HIDDEN), jnp.float32),
        scratch_types=[
            pltpu.VMEM((NCH, CH), jnp.int32),
            pltpu.VMEM((NCH, CH), jnp.int32),
            pltpu.VMEM((2, CH, HIDDEN), jnp.float32),
            pltpu.SemaphoreType.DMA,
            pltpu.SemaphoreType.DMA,
            pltpu.SemaphoreType.DMA,
            pltpu.SemaphoreType.DMA,
        ],
    )
    def k(dest_hbm, tok_hbm, hid_hbm, xs_hbm, dest_v, tok_v, rows_v,
          sem_g0, sem_g1, sem_s0, sem_s1):
        sems_g = (sem_g0, sem_g1)
        sems_s = (sem_s0, sem_s1)
        wid = lax.axis_index("s") * 2 + lax.axis_index("c")
        pltpu.sync_copy(dest_hbm.at[wid], dest_v)
        pltpu.sync_copy(tok_hbm.at[wid], tok_v)
        gathers = []
        scatters = [None, None]
        for c in range(NCH):
            if scatters[c % 2] is not None:
                scatters[c % 2].wait()
                scatters[c % 2] = None
            gathers.append(
                pltpu.async_copy(hid_hbm.at[tok_v.at[c]], rows_v.at[c % 2],
                                 sems_g[c % 2]))
            if c >= 1:
                gathers[c - 1].wait()
                scatters[(c - 1) % 2] = pltpu.async_copy(
                    rows_v.at[(c - 1) % 2], xs_hbm.at[dest_v.at[c - 1]],
                    sems_s[(c - 1) % 2])
        gathers[NCH - 1].wait()
        scatters[(NCH - 1) % 2] = pltpu.async_copy(
            rows_v.at[(NCH - 1) % 2], xs_hbm.at[dest_v.at[NCH - 1]],
            sems_s[(NCH - 1) % 2])
        scatters[0].wait()
        scatters[1].wait()

    return k(dest3, tok3, hidden_states)


def _ffn_body(be_ref, nr_ref, xs_ref, w1_ref, w2_ref, ys_ref):
    i = pl.program_id(0)

    @pl.when(i < nr_ref[0])
    def _():
        x = xs_ref[...].astype(jnp.bfloat16)
        gu = lax.dot_general(x, w1_ref[0].astype(jnp.bfloat16),
                             (((1,), (1,)), ((), ())),
                             preferred_element_type=jnp.float32)
        g = gu[:, :FFN]
        u = gu[:, FFN:]
        h = (g * jax.nn.sigmoid(g) * u).astype(jnp.bfloat16)
        ys_ref[...] = lax.dot_general(h, w2_ref[0].astype(jnp.bfloat16),
                                      (((1,), (1,)), ((), ())),
                                      preferred_element_type=jnp.float32)


def _ffn(xs, w1, w2, be, nr):
    grid_spec = pltpu.PrefetchScalarGridSpec(
        num_scalar_prefetch=2,
        grid=(NB,),
        in_specs=[
            pl.BlockSpec((BS, HIDDEN), lambda i, be, nr: (i, 0)),
            pl.BlockSpec((1, 2 * FFN, HIDDEN), lambda i, be, nr: (be[i], 0, 0)),
            pl.BlockSpec((1, HIDDEN, FFN), lambda i, be, nr: (be[i], 0, 0)),
        ],
        out_specs=pl.BlockSpec((BS, HIDDEN), lambda i, be, nr: (i, 0)),
    )
    return pl.pallas_call(
        _ffn_body,
        grid_spec=grid_spec,
        out_shape=jax.ShapeDtypeStruct((S_MAX, HIDDEN), jnp.float32),
        interpret=_INTERPRET,
    )(be, nr, xs, w1, w2)


def _sc_combine(ys, pos3, wb2d):
    mesh = plsc.VectorSubcoreMesh(core_axis_name="c", subcore_axis_name="s")

    @functools.partial(
        pl.kernel, mesh=mesh,
        out_type=jax.ShapeDtypeStruct((T, HIDDEN), jnp.float32),
        scratch_types=[
            pltpu.VMEM((NCH, CH), jnp.int32),
            pltpu.VMEM((TPW, 2 * 16), jnp.float32),
            pltpu.VMEM((2, CH, HIDDEN), jnp.float32),
            pltpu.VMEM((TCH, HIDDEN), jnp.float32),
            pltpu.SemaphoreType.DMA,
            pltpu.SemaphoreType.DMA,
        ],
    )
    def k(ys_hbm, pos_hbm, wb_hbm, out_hbm, pos_v, wb_v, buf_v, out_v,
          sem0, sem1):
        sems = (sem0, sem1)
        wid = lax.axis_index("s") * 2 + lax.axis_index("c")
        pltpu.sync_copy(pos_hbm.at[wid], pos_v)
        pltpu.sync_copy(wb_hbm.at[pl.ds(wid * TPW, TPW)], wb_v)

        def compute(c):
            buf = buf_v.at[c % 2]
            for j in range(TCH):
                w0 = wb_v[c * TCH + j, 0:16]
                w1 = wb_v[c * TCH + j, 16:32]

                def body(v, carry, j=j, w0=w0, w1=w1, buf=buf):
                    for u in range(4):
                        sl = pl.ds(v * 64 + u * 16, 16)
                        out_v[j, sl] = buf[2 * j, sl] * w0 + \
                            buf[2 * j + 1, sl] * w1
                    return carry

                lax.fori_loop(0, HIDDEN // 64, body, 0)
            pltpu.sync_copy(out_v,
                            out_hbm.at[pl.ds(wid * TPW + c * TCH, TCH)])

        copies = []
        for c in range(NCH):
            copies.append(
                pltpu.async_copy(ys_hbm.at[pos_v.at[c]], buf_v.at[c % 2],
                                 sems[c % 2]))
            if c >= 1:
                copies[c - 1].wait()
                compute(c - 1)
        copies[NCH - 1].wait()
        compute(NCH - 1)

    return k(ys, pos3, wb2d)


def kernel(hidden_states, gate_w, w1, w2):
    i32 = jnp.int32
    ri = lax.broadcasted_iota(i32, (BT, BT), 0)
    ci = lax.broadcasted_iota(i32, (BT, BT), 1)
    tri = (ci <= ri).astype(jnp.bfloat16)

    router_logits, ids, wts, wb2d, rank, counts8 = _router(
        hidden_states, gate_w, tri)
    counts = counts8[0]

    padded = ((counts + BS - 1) // BS) * BS
    pend = jnp.cumsum(padded)
    poff = pend - padded
    nr = (pend[-1] // BS).astype(i32)
    bstarts = jnp.arange(NB, dtype=i32) * BS
    be_raw = jnp.minimum(jnp.sum(pend[None, :] <= bstarts[:, None], axis=1),
                         E - 1).astype(i32)
    be = jnp.where(jnp.arange(NB, dtype=i32) < nr, be_raw,
                   be_raw[jnp.maximum(nr - 1, 0)])

    dest = poff[ids.reshape(-1)] + rank.reshape(-1)
    dest3 = dest.reshape(NW, NCH, CH)
    tok3 = (jnp.arange(S, dtype=i32) // TOPK).reshape(NW, NCH, CH)

    xs = _sc_dispatch(dest3, tok3, hidden_states)
    ys = _ffn(xs, w1, w2, be, nr[None])
    final = _sc_combine(ys, dest3, wb2d)
    return final, router_logits


# speedup vs baseline: 1.2920x; 1.2920x over previous
import functools

import jax
import jax.numpy as jnp
from jax import lax
from jax.experimental import pallas as pl
from jax.experimental.pallas import tpu as pltpu
from jax.experimental.pallas import tpu_sc as plsc

HIDDEN = 1024
FFN = 1024
E = 64
TOPK = 2
T = 4096

S = T * TOPK
BS = 128
S_MAX = S + E * BS
NB = S_MAX // BS
BT = 512

NW = 32
APW = S // NW
NCH = 8
CH = APW // NCH
TPW = T // NW
TCH = CH // TOPK

_INTERPRET = False


def _router_body(x_ref, gw_ref, tri_ref, logits_ref, ids_ref, wts_ref,
                 wb_ref, rank_ref, counts_ref, carry_ref):
    i = pl.program_id(0)

    @pl.when(i == 0)
    def _():
        carry_ref[...] = jnp.zeros_like(carry_ref)

    x = x_ref[...]
    logits = lax.dot_general(x, gw_ref[...], (((1,), (1,)), ((), ())),
                             preferred_element_type=jnp.float32)
    logits_ref[...] = logits
    m = jnp.max(logits, axis=-1, keepdims=True)
    p = jnp.exp(logits - m)
    p = p / jnp.sum(p, axis=-1, keepdims=True)
    iot = lax.broadcasted_iota(jnp.int32, p.shape, 1)
    m0 = jnp.max(p, axis=-1, keepdims=True)
    i0 = jnp.min(jnp.where(p == m0, iot, E), axis=-1, keepdims=True)
    p2 = jnp.where(iot == i0, -jnp.inf, p)
    m1 = jnp.max(p2, axis=-1, keepdims=True)
    i1 = jnp.min(jnp.where(p2 == m1, iot, E), axis=-1, keepdims=True)
    s = m0 + m1
    ids_ref[...] = jnp.concatenate([i0, i1], axis=-1)
    wts_ref[...] = jnp.concatenate([m0 / s, m1 / s], axis=-1)
    wb_ref[...] = jnp.concatenate(
        [jnp.broadcast_to(m0 / s, (BT, 16)),
         jnp.broadcast_to(m1 / s, (BT, 16))], axis=-1)

    oh0 = (i0 == iot).astype(jnp.int32)
    oh1 = (i1 == iot).astype(jnp.int32)
    c01 = lax.dot_general(tri_ref[...], (oh0 + oh1).astype(jnp.bfloat16),
                          (((1,), (0,)), ((), ())),
                          preferred_element_type=jnp.float32
                          ).astype(jnp.int32)
    excl = c01 - oh0 - oh1
    carry = carry_ref[0:1, :]
    r0 = jnp.sum(oh0 * (excl + carry), axis=1, keepdims=True)
    r1 = jnp.sum(oh1 * (excl + oh0 + carry), axis=1, keepdims=True)
    rank_ref[...] = jnp.concatenate([r0, r1], axis=-1)
    carry_new = carry + c01[-1:, :]
    carry_ref[0:1, :] = carry_new
    counts_ref[...] = jnp.broadcast_to(carry_new, counts_ref.shape)


def _router(hidden_states, gate_w, tri):
    return pl.pallas_call(
        _router_body,
        grid=(T // BT,),
        in_specs=[
            pl.BlockSpec((BT, HIDDEN), lambda i: (i, 0)),
            pl.BlockSpec((E, HIDDEN), lambda i: (0, 0)),
            pl.BlockSpec((BT, BT), lambda i: (0, 0)),
        ],
        out_specs=[
            pl.BlockSpec((BT, E), lambda i: (i, 0)),
            pl.BlockSpec((BT, TOPK), lambda i: (i, 0)),
            pl.BlockSpec((BT, TOPK), lambda i: (i, 0)),
            pl.BlockSpec((BT, 2 * 16), lambda i: (i, 0)),
            pl.BlockSpec((BT, TOPK), lambda i: (i, 0)),
            pl.BlockSpec((8, E), lambda i: (0, 0)),
        ],
        out_shape=[
            jax.ShapeDtypeStruct((T, E), jnp.float32),
            jax.ShapeDtypeStruct((T, TOPK), jnp.int32),
            jax.ShapeDtypeStruct((T, TOPK), jnp.float32),
            jax.ShapeDtypeStruct((T, 2 * 16), jnp.float32),
            jax.ShapeDtypeStruct((T, TOPK), jnp.int32),
            jax.ShapeDtypeStruct((8, E), jnp.int32),
        ],
        scratch_shapes=[pltpu.VMEM((8, E), jnp.int32)],
        interpret=_INTERPRET,
    )(hidden_states, gate_w, tri)


def _sc_dispatch(dest3, tok3, hidden_states):
    mesh = plsc.VectorSubcoreMesh(core_axis_name="c", subcore_axis_name="s")

    @functools.partial(
        pl.kernel, mesh=mesh,
        out_type=jax.ShapeDtypeStruct((S_MAX, HIDDEN), jnp.float32),
        scratch_types=[
            pltpu.VMEM((NCH, CH), jnp.int32),
            pltpu.VMEM((NCH, CH), jnp.int32),
            pltpu.VMEM((2, CH, HIDDEN), jnp.float32),
            pltpu.SemaphoreType.DMA,
            pltpu.SemaphoreType.DMA,
            pltpu.SemaphoreType.DMA,
            pltpu.SemaphoreType.DMA,
        ],
    )
    def k(dest_hbm, tok_hbm, hid_hbm, xs_hbm, dest_v, tok_v, rows_v,
          sem_g0, sem_g1, sem_s0, sem_s1):
        sems_g = (sem_g0, sem_g1)
        sems_s = (sem_s0, sem_s1)
        wid = lax.axis_index("s") * 2 + lax.axis_index("c")
        pltpu.sync_copy(dest_hbm.at[wid], dest_v)
        pltpu.sync_copy(tok_hbm.at[wid], tok_v)
        gathers = []
        scatters = [None, None]
        for c in range(NCH):
            if scatters[c % 2] is not None:
                scatters[c % 2].wait()
                scatters[c % 2] = None
            gathers.append(
                pltpu.async_copy(hid_hbm.at[tok_v.at[c]], rows_v.at[c % 2],
                                 sems_g[c % 2]))
            if c >= 1:
                gathers[c - 1].wait()
                scatters[(c - 1) % 2] = pltpu.async_copy(
                    rows_v.at[(c - 1) % 2], xs_hbm.at[dest_v.at[c - 1]],
                    sems_s[(c - 1) % 2])
        gathers[NCH - 1].wait()
        scatters[(NCH - 1) % 2] = pltpu.async_copy(
            rows_v.at[(NCH - 1) % 2], xs_hbm.at[dest_v.at[NCH - 1]],
            sems_s[(NCH - 1) % 2])
        scatters[0].wait()
        scatters[1].wait()

    return k(dest3, tok3, hidden_states)


def _ffn_body(be_ref, nr_ref, xs_ref, w1_ref, w2_ref, ys_ref):
    i = pl.program_id(0)

    @pl.when(i < nr_ref[0])
    def _():
        x = xs_ref[...].astype(jnp.bfloat16)
        gu = lax.dot_general(x, w1_ref[0].astype(jnp.bfloat16),
                             (((1,), (1,)), ((), ())),
                             preferred_element_type=jnp.float32)
        g = gu[:, :FFN]
        u = gu[:, FFN:]
        h = (g * jax.nn.sigmoid(g) * u).astype(jnp.bfloat16)
        ys_ref[...] = lax.dot_general(h, w2_ref[0].astype(jnp.bfloat16),
                                      (((1,), (1,)), ((), ())),
                                      preferred_element_type=jnp.float32)


def _ffn(xs, w1, w2, be, nr):
    grid_spec = pltpu.PrefetchScalarGridSpec(
        num_scalar_prefetch=2,
        grid=(NB,),
        in_specs=[
            pl.BlockSpec((BS, HIDDEN),
                         lambda i, be, nr: (jnp.minimum(i, nr[0] - 1), 0)),
            pl.BlockSpec((1, 2 * FFN, HIDDEN), lambda i, be, nr: (be[i], 0, 0)),
            pl.BlockSpec((1, HIDDEN, FFN), lambda i, be, nr: (be[i], 0, 0)),
        ],
        out_specs=pl.BlockSpec(
            (BS, HIDDEN), lambda i, be, nr: (jnp.minimum(i, nr[0] - 1), 0)),
    )
    return pl.pallas_call(
        _ffn_body,
        grid_spec=grid_spec,
        out_shape=jax.ShapeDtypeStruct((S_MAX, HIDDEN), jnp.float32),
        interpret=_INTERPRET,
    )(be, nr, xs, w1, w2)


def _sc_combine(ys, pos3, wb2d):
    mesh = plsc.VectorSubcoreMesh(core_axis_name="c", subcore_axis_name="s")

    @functools.partial(
        pl.kernel, mesh=mesh,
        out_type=jax.ShapeDtypeStruct((T, HIDDEN), jnp.float32),
        scratch_types=[
            pltpu.VMEM((NCH, CH), jnp.int32),
            pltpu.VMEM((TPW, 2 * 16), jnp.float32),
            pltpu.VMEM((2, CH, HIDDEN), jnp.float32),
            pltpu.VMEM((TCH, HIDDEN), jnp.float32),
            pltpu.SemaphoreType.DMA,
            pltpu.SemaphoreType.DMA,
        ],
    )
    def k(ys_hbm, pos_hbm, wb_hbm, out_hbm, pos_v, wb_v, buf_v, out_v,
          sem0, sem1):
        sems = (sem0, sem1)
        wid = lax.axis_index("s") * 2 + lax.axis_index("c")
        pltpu.sync_copy(pos_hbm.at[wid], pos_v)
        pltpu.sync_copy(wb_hbm.at[pl.ds(wid * TPW, TPW)], wb_v)

        def compute(c):
            buf = buf_v.at[c % 2]
            for j in range(TCH):
                w0 = wb_v[c * TCH + j, 0:16]
                w1 = wb_v[c * TCH + j, 16:32]

                def body(v, carry, j=j, w0=w0, w1=w1, buf=buf):
                    for u in range(4):
                        sl = pl.ds(v * 64 + u * 16, 16)
                        out_v[j, sl] = buf[2 * j, sl] * w0 + \
                            buf[2 * j + 1, sl] * w1
                    return carry

                lax.fori_loop(0, HIDDEN // 64, body, 0)
            pltpu.sync_copy(out_v,
                            out_hbm.at[pl.ds(wid * TPW + c * TCH, TCH)])

        copies = []
        for c in range(NCH):
            copies.append(
                pltpu.async_copy(ys_hbm.at[pos_v.at[c]], buf_v.at[c % 2],
                                 sems[c % 2]))
            if c >= 1:
                copies[c - 1].wait()
                compute(c - 1)
        copies[NCH - 1].wait()
        compute(NCH - 1)

    return k(ys, pos3, wb2d)


def kernel(hidden_states, gate_w, w1, w2):
    i32 = jnp.int32
    ri = lax.broadcasted_iota(i32, (BT, BT), 0)
    ci = lax.broadcasted_iota(i32, (BT, BT), 1)
    tri = (ci <= ri).astype(jnp.bfloat16)

    router_logits, ids, wts, wb2d, rank, counts8 = _router(
        hidden_states, gate_w, tri)
    counts = counts8[0]

    padded = ((counts + BS - 1) // BS) * BS
    pend = jnp.cumsum(padded)
    poff = pend - padded
    nr = (pend[-1] // BS).astype(i32)
    bstarts = jnp.arange(NB, dtype=i32) * BS
    be_raw = jnp.minimum(jnp.sum(pend[None, :] <= bstarts[:, None], axis=1),
                         E - 1).astype(i32)
    be = jnp.where(jnp.arange(NB, dtype=i32) < nr, be_raw,
                   be_raw[jnp.maximum(nr - 1, 0)])

    dest = poff[ids.reshape(-1)] + rank.reshape(-1)
    dest3 = dest.reshape(NW, NCH, CH)
    tok3 = (jnp.arange(S, dtype=i32) // TOPK).reshape(NW, NCH, CH)

    xs = _sc_dispatch(dest3, tok3, hidden_states)
    ys = _ffn(xs, w1, w2, be, nr[None])
    final = _sc_combine(ys, dest3, wb2d)
    return final, router_logits
